# baseline (speedup 1.0000x reference)
import jax
import jax.numpy as jnp
from jax.experimental import pallas as pl
from jax.experimental.pallas import tpu as pltpu


def _interp_matrix(n_in: int, n_out: int) -> jnp.ndarray:
    if n_out == 1:
        src = jnp.zeros((1,), dtype=jnp.float32)
    else:
        src = jnp.arange(n_out, dtype=jnp.float32) * (n_in - 1) / (n_out - 1)
    i0 = jnp.clip(jnp.floor(src).astype(jnp.int32), 0, n_in - 1)
    i1 = jnp.clip(i0 + 1, 0, n_in - 1)
    frac = src - i0.astype(jnp.float32)
    cols = jnp.arange(n_in, dtype=jnp.int32)
    return ((cols[None, :] == i0[:, None]).astype(jnp.float32) * (1.0 - frac)[:, None]
            + (cols[None, :] == i1[:, None]).astype(jnp.float32) * frac[:, None])


def _upsample_tile_kernel(m_ref, x_ref, o_ref):
    r = jnp.dot(x_ref[...], m_ref[...], preferred_element_type=jnp.float32)
    o_ref[...] = r.astype(jnp.bfloat16)


def kernel(x):
    n, c, d, h, w = x.shape
    h_out, w_out = 2 * h, 2 * w
    b = n * c * d
    hw, hw_out = h * w, h_out * w_out

    a_h = _interp_matrix(h, h_out)
    a_w = _interp_matrix(w, w_out)
    m = jnp.kron(a_h.T, a_w.T).astype(jnp.bfloat16)

    tb = 8192
    grid = b // tb

    x2 = x.astype(jnp.bfloat16).reshape(b, hw)

    out = pl.pallas_call(
        _upsample_tile_kernel,
        out_shape=jax.ShapeDtypeStruct((b, hw_out), jnp.bfloat16),
        grid=(grid,),
        in_specs=[
            pl.BlockSpec((hw, hw_out), lambda i: (0, 0)),
            pl.BlockSpec((tb, hw), lambda i: (i, 0)),
        ],
        out_specs=pl.BlockSpec((tb, hw_out), lambda i: (i, 0)),
        compiler_params=pltpu.CompilerParams(
            dimension_semantics=("parallel",),
            vmem_limit_bytes=64 << 20,
        ),
    )(m, x2)

    return out.reshape(n, c, d, h_out, w_out).astype(jnp.float32)


# speedup vs baseline: 1.0029x; 1.0029x over previous
import jax
import jax.numpy as jnp
from jax.experimental import pallas as pl
from jax.experimental.pallas import tpu as pltpu


def _interp_matrix(n_in: int, n_out: int) -> jnp.ndarray:
    if n_out == 1:
        src = jnp.zeros((1,), dtype=jnp.float32)
    else:
        src = jnp.arange(n_out, dtype=jnp.float32) * (n_in - 1) / (n_out - 1)
    i0 = jnp.clip(jnp.floor(src).astype(jnp.int32), 0, n_in - 1)
    i1 = jnp.clip(i0 + 1, 0, n_in - 1)
    frac = src - i0.astype(jnp.float32)
    cols = jnp.arange(n_in, dtype=jnp.int32)
    return ((cols[None, :] == i0[:, None]).astype(jnp.float32) * (1.0 - frac)[:, None]
            + (cols[None, :] == i1[:, None]).astype(jnp.float32) * frac[:, None])


def _upsample_tile_kernel(m_ref, x_ref, o_ref):
    r = jnp.dot(x_ref[...], m_ref[...], preferred_element_type=jnp.float32)
    o_ref[...] = r.astype(jnp.bfloat16)


def kernel(x):
    n, c, d, h, w = x.shape
    h_out, w_out = 2 * h, 2 * w
    b = n * c * d
    hw, hw_out = h * w, h_out * w_out

    a_h = _interp_matrix(h, h_out)
    a_w = _interp_matrix(w, w_out)
    m = jnp.kron(a_h.T, a_w.T).astype(jnp.bfloat16)

    tb = 4096
    while tb > 8 and b % tb:
        tb //= 2
    b_padded = pl.cdiv(b, tb) * tb

    x2 = x.astype(jnp.bfloat16).reshape(b, hw)
    if b_padded != b:
        x2 = jnp.pad(x2, ((0, b_padded - b), (0, 0)))

    out2 = pl.pallas_call(
        _upsample_tile_kernel,
        out_shape=jax.ShapeDtypeStruct((b_padded, hw_out), jnp.bfloat16),
        grid=(b_padded // tb,),
        in_specs=[
            pl.BlockSpec((hw, hw_out), lambda i: (0, 0)),
            pl.BlockSpec((tb, hw), lambda i: (i, 0)),
        ],
        out_specs=pl.BlockSpec((tb, hw_out), lambda i: (i, 0)),
        compiler_params=pltpu.CompilerParams(
            dimension_semantics=("parallel",),
            vmem_limit_bytes=64 << 20,
        ),
    )(m, x2)

    return out2[:b].reshape(n, c, d, h_out, w_out).astype(jnp.float32)
